# baseline (speedup 1.0000x reference)
import functools

import jax
import jax.numpy as jnp
from jax import lax
from jax.experimental import pallas as pl
from jax.experimental.pallas import tpu as pltpu
from jax.experimental.pallas import tpu_sc as plsc

D_MODEL = 64
LANES = 16
NUM_WORKERS = 32
BATCH = 1024
SEQ_LEN = 200
QUARTER = BATCH // 4
CB = 2048


def _positional_encoding(max_len, d_model):
    depth = d_model // 2
    positions = jnp.arange(max_len, dtype=jnp.float32)[:, None]
    depths = jnp.arange(depth, dtype=jnp.float32)[None, :] / depth
    angle_rates = 1.0 / (10000.0 ** depths)
    angle_rads = positions * angle_rates
    return jnp.concatenate(
        [jnp.sin(angle_rads), jnp.cos(angle_rads)], axis=-1
    ).astype(jnp.float32)


def _tr_body(tt_ref, out_ref):
    x = tt_ref[...]
    scale = jnp.float32(8.0)
    y0 = jnp.transpose(x[:, : CB // 2]) * scale
    y1 = jnp.transpose(x[:, CB // 2 :]) * scale
    out_ref[...] = jnp.concatenate([y0, y1], axis=1)


def _pack_table(tt, vocab):
    grid = (vocab + CB - 1) // CB
    return pl.pallas_call(
        _tr_body,
        grid=(grid,),
        in_specs=[pl.BlockSpec((D_MODEL, CB), lambda i: (0, i))],
        out_specs=pl.BlockSpec((CB // 2, 128), lambda i: (i, 0)),
        out_shape=jax.ShapeDtypeStruct((grid * (CB // 2), 128), jnp.float32),
    )(tt)


def _gather_body(
    seq_ref, table_ref, pos_ref, out_ref,
    idx0, idx1, pidx0, pidx1, g0, g1, t0, t1, p0, p1,
    gs0, gs1, ss0, ss1,
):
    idxs = (idx0, idx1)
    pidxs = (pidx0, pidx1)
    gbufs = (g0, g1)
    tbufs = (t0, t1)
    pbufs = (p0, p1)
    gsems = (gs0, gs1)
    ssems = (ss0, ss1)

    nc = 2
    wid = lax.axis_index("s") * nc + lax.axis_index("c")
    chunks_per_w = (SEQ_LEN * 4) // NUM_WORKERS
    c_base = wid * chunks_per_w
    iota = lax.iota(jnp.int32, LANES)

    def stage(k, r):
        c = c_base + k
        tc = c >> 2
        qc = c & 3
        pltpu.sync_copy(
            seq_ref.at[tc, pl.ds(qc * QUARTER, QUARTER)], idxs[r]
        )
        pltpu.sync_copy(pos_ref.at[tc], pbufs[r])
        for i in range(QUARTER // LANES):
            sl = pl.ds(i * LANES, LANES)
            v16 = idxs[r][sl]
            pidxs[r][sl] = lax.shift_left(
                lax.shift_right_logical(v16, 11), 10
            ) | (v16 & (CB // 2 - 1))
        for h in range(2):
            pltpu.async_copy(
                table_ref.at[pidxs[r].at[pl.ds(h * 128, 128)]],
                gbufs[r].at[pl.ds(h * 128, 128)],
                gsems[r],
            )

    def wait_gathers(r):
        for h in range(2):
            pltpu.make_async_copy(
                table_ref.at[pidxs[r].at[pl.ds(h * 128, 128)]],
                gbufs[r].at[pl.ds(h * 128, 128)],
                gsems[r],
            ).wait()

    def scatter_copy(k, r):
        c = c_base + k
        tc = c >> 2
        qc = c & 3
        return pltpu.make_async_copy(
            tbufs[r],
            out_ref.at[tc, :, pl.ds(qc * QUARTER, QUARTER)],
            ssems[r],
        )

    colv = [(iota + k) & 15 for k in range(LANES)]

    def compute(r):
        gbuf = gbufs[r]
        tbuf = tbufs[r]
        pbuf = pbufs[r]

        @pl.loop(0, QUARTER // LANES)
        def _bc_loop(bc):
            sl = pl.ds(bc * LANES, LANES)
            v16 = idxs[r][sl]
            rowv = iota + bc * LANES
            hvec = lax.shift_left(lax.shift_right_logical(v16, 10) & 1, 6)
            for fb in range(D_MODEL // LANES):
                for k in range(LANES):
                    featv = colv[k] + (fb * LANES)
                    gval = plsc.load_gather(gbuf, [rowv, featv + hvec])
                    pval = plsc.load_gather(
                        pbuf, [lax.shift_left(featv, 4) + iota]
                    )
                    plsc.store_scatter(tbuf, [featv, rowv], gval + pval)

    stage(0, 0)
    stage(1, 1)
    wait_gathers(0)
    compute(0)
    scatter_copy(0, 0).start()

    stage(2, 0)
    wait_gathers(1)
    compute(1)
    scatter_copy(1, 1).start()

    @pl.loop(0, 11)
    def _main(p):
        for rr in range(2):
            k = 2 + 2 * p + rr
            q = rr
            stage(k + 1, 1 - q)
            wait_gathers(q)
            scatter_copy(k - 2, q).wait()
            compute(q)
            scatter_copy(k, q).start()

    wait_gathers(0)
    scatter_copy(22, 0).wait()
    compute(0)
    scatter_copy(24, 0).start()
    scatter_copy(23, 1).wait()
    scatter_copy(24, 0).wait()


def kernel(sequences, embedding_table):
    batch, seq_len = sequences.shape
    vocab, d_model = embedding_table.shape
    assert (batch, seq_len, d_model) == (BATCH, SEQ_LEN, D_MODEL)

    tt = jnp.transpose(embedding_table)
    table_p = _pack_table(tt, vocab)

    seq_t = jnp.transpose(sequences).astype(jnp.int32)
    pos = _positional_encoding(seq_len, d_model)
    pos_b = jnp.broadcast_to(
        pos[:, :, None], (seq_len, d_model, LANES)
    ).reshape(seq_len, d_model * LANES)

    mesh = plsc.VectorSubcoreMesh(core_axis_name="c", subcore_axis_name="s")
    out_p = pl.kernel(
        _gather_body,
        out_type=jax.ShapeDtypeStruct((seq_len, d_model, batch), jnp.float32),
        mesh=mesh,
        compiler_params=pltpu.CompilerParams(needs_layout_passes=False),
        scratch_types=[
            pltpu.VMEM((QUARTER,), jnp.int32),
            pltpu.VMEM((QUARTER,), jnp.int32),
            pltpu.VMEM((QUARTER,), jnp.int32),
            pltpu.VMEM((QUARTER,), jnp.int32),
            pltpu.VMEM((QUARTER, 128), jnp.float32),
            pltpu.VMEM((QUARTER, 128), jnp.float32),
            pltpu.VMEM((D_MODEL, QUARTER), jnp.float32),
            pltpu.VMEM((D_MODEL, QUARTER), jnp.float32),
            pltpu.VMEM((D_MODEL * LANES,), jnp.float32),
            pltpu.VMEM((D_MODEL * LANES,), jnp.float32),
        ]
        + [pltpu.SemaphoreType.DMA for _ in range(4)],
    )(seq_t, table_p, pos_b)
    return jnp.transpose(out_p, (2, 0, 1))


# speedup vs baseline: 1.2126x; 1.2126x over previous
import functools

import jax
import jax.numpy as jnp
from jax import lax
from jax.experimental import pallas as pl
from jax.experimental.pallas import tpu as pltpu
from jax.experimental.pallas import tpu_sc as plsc

D_MODEL = 64
LANES = 16
NUM_WORKERS = 32
BATCH = 1024
SEQ_LEN = 200
QUARTER = BATCH // 4
CB = 2048


def _positional_encoding(max_len, d_model):
    depth = d_model // 2
    positions = jnp.arange(max_len, dtype=jnp.float32)[:, None]
    depths = jnp.arange(depth, dtype=jnp.float32)[None, :] / depth
    angle_rates = 1.0 / (10000.0 ** depths)
    angle_rads = positions * angle_rates
    return jnp.concatenate(
        [jnp.sin(angle_rads), jnp.cos(angle_rads)], axis=-1
    ).astype(jnp.float32)


def _tr_body(tt_ref, out_ref):
    x = tt_ref[...]
    scale = jnp.float32(8.0)
    y0 = jnp.transpose(x[:, : CB // 2]) * scale
    y1 = jnp.transpose(x[:, CB // 2 :]) * scale
    out_ref[...] = jnp.concatenate([y0, y1], axis=1)


def _pack_table(tt, vocab):
    grid = (vocab + CB - 1) // CB
    return pl.pallas_call(
        _tr_body,
        grid=(grid,),
        in_specs=[pl.BlockSpec((D_MODEL, CB), lambda i: (0, i))],
        out_specs=pl.BlockSpec((CB // 2, 128), lambda i: (i, 0)),
        out_shape=jax.ShapeDtypeStruct((grid * (CB // 2), 128), jnp.float32),
    )(tt)


def _gather_body(
    seq_ref, table_ref, pos_ref, out_ref,
    idx0, idx1, pidx0, pidx1, g0, g1, t0, t1, p0, p1,
    gs0, gs1, ss0, ss1,
):
    idxs = (idx0, idx1)
    pidxs = (pidx0, pidx1)
    gbufs = (g0, g1)
    tbufs = (t0, t1)
    pbufs = (p0, p1)
    gsems = (gs0, gs1)
    ssems = (ss0, ss1)

    nc = 2
    wid = lax.axis_index("s") * nc + lax.axis_index("c")
    chunks_per_w = (SEQ_LEN * 4) // NUM_WORKERS
    c_base = wid * chunks_per_w
    iota = lax.iota(jnp.int32, LANES)

    def stage(k, r):
        c = c_base + k
        tc = c >> 2
        qc = c & 3
        pltpu.sync_copy(
            seq_ref.at[tc, pl.ds(qc * QUARTER, QUARTER)], idxs[r]
        )
        pltpu.sync_copy(pos_ref.at[tc], pbufs[r])
        for i in range(QUARTER // LANES):
            sl = pl.ds(i * LANES, LANES)
            v16 = idxs[r][sl]
            pidxs[r][sl] = lax.shift_left(
                lax.shift_right_logical(v16, 11), 10
            ) | (v16 & (CB // 2 - 1))
        for h in range(2):
            pltpu.async_copy(
                table_ref.at[pidxs[r].at[pl.ds(h * 128, 128)]],
                gbufs[r].at[pl.ds(h * 128, 128)],
                gsems[r],
            )

    def wait_gathers(r):
        for h in range(2):
            pltpu.make_async_copy(
                table_ref.at[pidxs[r].at[pl.ds(h * 128, 128)]],
                gbufs[r].at[pl.ds(h * 128, 128)],
                gsems[r],
            ).wait()

    def scatter_copy(k, r):
        c = c_base + k
        tc = c >> 2
        qc = c & 3
        return pltpu.make_async_copy(
            tbufs[r],
            out_ref.at[tc, :, pl.ds(qc * QUARTER, QUARTER)],
            ssems[r],
        )

    colv = [(iota + k) & 15 for k in range(LANES)]

    def compute(r):
        gbuf = gbufs[r]
        tbuf = tbufs[r]
        pbuf = pbufs[r]

        @pl.loop(0, QUARTER // LANES)
        def _bc_loop(bc):
            sl = pl.ds(bc * LANES, LANES)
            v16 = idxs[r][sl]
            rowv = iota + bc * LANES
            hvec = lax.shift_left(lax.shift_right_logical(v16, 10) & 1, 6)
            for fb in range(D_MODEL // LANES):
                base = fb * LANES
                for kh in range(0, LANES, 8):
                    ks = range(kh, kh + 8)
                    featvs = [colv[k] + base for k in ks]
                    gvals = [
                        plsc.load_gather(gbuf, [rowv, fv + hvec])
                        for fv in featvs
                    ]
                    pvals = [
                        plsc.load_gather(
                            pbuf, [lax.shift_left(fv, 4) + iota]
                        )
                        for fv in featvs
                    ]
                    for fv, gv, pv in zip(featvs, gvals, pvals):
                        plsc.store_scatter(tbuf, [fv, rowv], gv + pv)

    stage(0, 0)
    stage(1, 1)
    wait_gathers(0)
    compute(0)
    scatter_copy(0, 0).start()

    stage(2, 0)
    wait_gathers(1)
    compute(1)
    scatter_copy(1, 1).start()

    @pl.loop(0, 11)
    def _main(p):
        for rr in range(2):
            k = 2 + 2 * p + rr
            q = rr
            stage(k + 1, 1 - q)
            wait_gathers(q)
            scatter_copy(k - 2, q).wait()
            compute(q)
            scatter_copy(k, q).start()

    wait_gathers(0)
    scatter_copy(22, 0).wait()
    compute(0)
    scatter_copy(24, 0).start()
    scatter_copy(23, 1).wait()
    scatter_copy(24, 0).wait()


def kernel(sequences, embedding_table):
    batch, seq_len = sequences.shape
    vocab, d_model = embedding_table.shape
    assert (batch, seq_len, d_model) == (BATCH, SEQ_LEN, D_MODEL)

    tt = jnp.transpose(embedding_table)
    table_p = _pack_table(tt, vocab)

    seq_t = jnp.transpose(sequences).astype(jnp.int32)
    pos = _positional_encoding(seq_len, d_model)
    pos_b = jnp.broadcast_to(
        pos[:, :, None], (seq_len, d_model, LANES)
    ).reshape(seq_len, d_model * LANES)

    mesh = plsc.VectorSubcoreMesh(core_axis_name="c", subcore_axis_name="s")
    out_p = pl.kernel(
        _gather_body,
        out_type=jax.ShapeDtypeStruct((seq_len, d_model, batch), jnp.float32),
        mesh=mesh,
        compiler_params=pltpu.CompilerParams(needs_layout_passes=False),
        scratch_types=[
            pltpu.VMEM((QUARTER,), jnp.int32),
            pltpu.VMEM((QUARTER,), jnp.int32),
            pltpu.VMEM((QUARTER,), jnp.int32),
            pltpu.VMEM((QUARTER,), jnp.int32),
            pltpu.VMEM((QUARTER, 128), jnp.float32),
            pltpu.VMEM((QUARTER, 128), jnp.float32),
            pltpu.VMEM((D_MODEL, QUARTER), jnp.float32),
            pltpu.VMEM((D_MODEL, QUARTER), jnp.float32),
            pltpu.VMEM((D_MODEL * LANES,), jnp.float32),
            pltpu.VMEM((D_MODEL * LANES,), jnp.float32),
        ]
        + [pltpu.SemaphoreType.DMA for _ in range(4)],
    )(seq_t, table_p, pos_b)
    return jnp.transpose(out_p, (2, 0, 1))


# speedup vs baseline: 1.5439x; 1.2732x over previous
import functools

import jax
import jax.numpy as jnp
from jax import lax
from jax.experimental import pallas as pl
from jax.experimental.pallas import tpu as pltpu
from jax.experimental.pallas import tpu_sc as plsc

D_MODEL = 64
LANES = 16
NUM_WORKERS = 32
BATCH = 1024
SEQ_LEN = 200
QUARTER = BATCH // 4
CB = 4096
CB_SH = 12


def _positional_encoding(max_len, d_model):
    depth = d_model // 2
    positions = jnp.arange(max_len, dtype=jnp.float32)[:, None]
    depths = jnp.arange(depth, dtype=jnp.float32)[None, :] / depth
    angle_rates = 1.0 / (10000.0 ** depths)
    angle_rads = positions * angle_rates
    return jnp.concatenate(
        [jnp.sin(angle_rads), jnp.cos(angle_rads)], axis=-1
    ).astype(jnp.float32)


def _tr_body(tt_ref, out_ref):
    x = tt_ref[...]
    scale = jnp.float32(8.0)
    y0 = jnp.transpose(x[:, : CB // 2]) * scale
    y1 = jnp.transpose(x[:, CB // 2 :]) * scale
    out_ref[...] = jnp.concatenate([y0, y1], axis=1)


def _pack_table(tt, vocab):
    grid = (vocab + CB - 1) // CB
    return pl.pallas_call(
        _tr_body,
        grid=(grid,),
        in_specs=[pl.BlockSpec((D_MODEL, CB), lambda i: (0, i))],
        out_specs=pl.BlockSpec((CB // 2, 128), lambda i: (i, 0)),
        out_shape=jax.ShapeDtypeStruct((grid * (CB // 2), 128), jnp.float32),
    )(tt)


def _gather_body(
    seq_ref, table_ref, pos_ref, out_ref,
    idx0, idx1, pidx0, pidx1, g0, g1, t0, t1, p0, p1,
    gs0, gs1, ss0, ss1,
):
    idxs = (idx0, idx1)
    pidxs = (pidx0, pidx1)
    gbufs = (g0, g1)
    tbufs = (t0, t1)
    pbufs = (p0, p1)
    gsems = (gs0, gs1)
    ssems = (ss0, ss1)

    nc = 2
    wid = lax.axis_index("s") * nc + lax.axis_index("c")
    chunks_per_w = (SEQ_LEN * 4) // NUM_WORKERS
    c_base = wid * chunks_per_w
    iota = lax.iota(jnp.int32, LANES)

    def stage(k, r):
        c = c_base + k
        tc = c >> 2
        qc = c & 3
        pltpu.sync_copy(
            seq_ref.at[tc, pl.ds(qc * QUARTER, QUARTER)], idxs[r]
        )
        pltpu.sync_copy(pos_ref.at[tc], pbufs[r])
        for i in range(QUARTER // LANES):
            sl = pl.ds(i * LANES, LANES)
            v16 = idxs[r][sl]
            pidxs[r][sl] = lax.shift_left(
                lax.shift_right_logical(v16, CB_SH), CB_SH - 1
            ) | (v16 & (CB // 2 - 1))
        for h in range(2):
            pltpu.async_copy(
                table_ref.at[pidxs[r].at[pl.ds(h * 128, 128)]],
                gbufs[r].at[pl.ds(h * 128, 128)],
                gsems[r],
            )

    def wait_gathers(r):
        for h in range(2):
            pltpu.make_async_copy(
                table_ref.at[pidxs[r].at[pl.ds(h * 128, 128)]],
                gbufs[r].at[pl.ds(h * 128, 128)],
                gsems[r],
            ).wait()

    def scatter_copy(k, r):
        c = c_base + k
        tc = c >> 2
        qc = c & 3
        return pltpu.make_async_copy(
            tbufs[r],
            out_ref.at[tc, :, pl.ds(qc * QUARTER, QUARTER)],
            ssems[r],
        )

    colv = [(iota + k) & 15 for k in range(LANES)]

    def compute(r):
        gbuf = gbufs[r]
        tbuf = tbufs[r]
        pbuf = pbufs[r]

        @pl.loop(0, QUARTER // LANES)
        def _bc_loop(bc):
            sl = pl.ds(bc * LANES, LANES)
            v16 = idxs[r][sl]
            rowv = iota + bc * LANES
            hvec = lax.shift_left(lax.shift_right_logical(v16, CB_SH - 1) & 1, 6)
            for fb in range(D_MODEL // LANES):
                base = fb * LANES
                for kh in range(0, LANES, 8):
                    ks = range(kh, kh + 8)
                    featvs = [colv[k] + base for k in ks]
                    gvals = [
                        plsc.load_gather(gbuf, [rowv, fv + hvec])
                        for fv in featvs
                    ]
                    pvals = [
                        plsc.load_gather(
                            pbuf, [lax.shift_left(fv, 4) + iota]
                        )
                        for fv in featvs
                    ]
                    for fv, gv, pv in zip(featvs, gvals, pvals):
                        plsc.store_scatter(tbuf, [fv, rowv], gv + pv)

    stage(0, 0)
    stage(1, 1)
    wait_gathers(0)
    compute(0)
    scatter_copy(0, 0).start()

    stage(2, 0)
    wait_gathers(1)
    compute(1)
    scatter_copy(1, 1).start()

    @pl.loop(0, 11)
    def _main(p):
        for rr in range(2):
            k = 2 + 2 * p + rr
            q = rr
            stage(k + 1, 1 - q)
            wait_gathers(q)
            scatter_copy(k - 2, q).wait()
            compute(q)
            scatter_copy(k, q).start()

    wait_gathers(0)
    scatter_copy(22, 0).wait()
    compute(0)
    scatter_copy(24, 0).start()
    scatter_copy(23, 1).wait()
    scatter_copy(24, 0).wait()


def kernel(sequences, embedding_table):
    batch, seq_len = sequences.shape
    vocab, d_model = embedding_table.shape
    assert (batch, seq_len, d_model) == (BATCH, SEQ_LEN, D_MODEL)

    tt = jnp.transpose(embedding_table)
    table_p = _pack_table(tt, vocab)

    seq_t = jnp.transpose(sequences).astype(jnp.int32)
    pos = _positional_encoding(seq_len, d_model)
    pos_b = jnp.broadcast_to(
        pos[:, :, None], (seq_len, d_model, LANES)
    ).reshape(seq_len, d_model * LANES)

    mesh = plsc.VectorSubcoreMesh(core_axis_name="c", subcore_axis_name="s")
    out_p = pl.kernel(
        _gather_body,
        out_type=jax.ShapeDtypeStruct((seq_len, d_model, batch), jnp.float32),
        mesh=mesh,
        compiler_params=pltpu.CompilerParams(needs_layout_passes=False),
        scratch_types=[
            pltpu.VMEM((QUARTER,), jnp.int32),
            pltpu.VMEM((QUARTER,), jnp.int32),
            pltpu.VMEM((QUARTER,), jnp.int32),
            pltpu.VMEM((QUARTER,), jnp.int32),
            pltpu.VMEM((QUARTER, 128), jnp.float32),
            pltpu.VMEM((QUARTER, 128), jnp.float32),
            pltpu.VMEM((D_MODEL, QUARTER), jnp.float32),
            pltpu.VMEM((D_MODEL, QUARTER), jnp.float32),
            pltpu.VMEM((D_MODEL * LANES,), jnp.float32),
            pltpu.VMEM((D_MODEL * LANES,), jnp.float32),
        ]
        + [pltpu.SemaphoreType.DMA for _ in range(4)],
    )(seq_t, table_p, pos_b)
    return jnp.transpose(out_p, (2, 0, 1))


# speedup vs baseline: 1.9883x; 1.2878x over previous
import functools

import jax
import jax.numpy as jnp
from jax import lax
from jax.experimental import pallas as pl
from jax.experimental.pallas import tpu as pltpu
from jax.experimental.pallas import tpu_sc as plsc

D_MODEL = 64
LANES = 16
NUM_WORKERS = 32
BATCH = 1024
SEQ_LEN = 200
QUARTER = BATCH // 4
CB = 16384
CB_SH = 14


def _positional_encoding(max_len, d_model):
    depth = d_model // 2
    positions = jnp.arange(max_len, dtype=jnp.float32)[:, None]
    depths = jnp.arange(depth, dtype=jnp.float32)[None, :] / depth
    angle_rates = 1.0 / (10000.0 ** depths)
    angle_rads = positions * angle_rates
    return jnp.concatenate(
        [jnp.sin(angle_rads), jnp.cos(angle_rads)], axis=-1
    ).astype(jnp.float32)


def _tr_body(tt_ref, out_ref):
    x = tt_ref[...]
    scale = jnp.float32(8.0)
    y0 = jnp.transpose(x[:, : CB // 2]) * scale
    y1 = jnp.transpose(x[:, CB // 2 :]) * scale
    out_ref[...] = jnp.concatenate([y0, y1], axis=1)


def _pack_table(tt, vocab):
    grid = (vocab + CB - 1) // CB
    return pl.pallas_call(
        _tr_body,
        grid=(grid,),
        in_specs=[pl.BlockSpec((D_MODEL, CB), lambda i: (0, i))],
        out_specs=pl.BlockSpec((CB // 2, 128), lambda i: (i, 0)),
        out_shape=jax.ShapeDtypeStruct((grid * (CB // 2), 128), jnp.float32),
    )(tt)


def _gather_body(
    seq_ref, table_ref, pos_ref, out_ref,
    idx0, idx1, pidx0, pidx1, g0, g1, t0, t1, p0, p1,
    gs0, gs1, ss0, ss1,
):
    idxs = (idx0, idx1)
    pidxs = (pidx0, pidx1)
    gbufs = (g0, g1)
    tbufs = (t0, t1)
    pbufs = (p0, p1)
    gsems = (gs0, gs1)
    ssems = (ss0, ss1)

    nc = 2
    wid = lax.axis_index("s") * nc + lax.axis_index("c")
    chunks_per_w = (SEQ_LEN * 4) // NUM_WORKERS
    c_base = wid * chunks_per_w
    iota = lax.iota(jnp.int32, LANES)

    def stage(k, r):
        c = c_base + k
        tc = c >> 2
        qc = c & 3
        pltpu.sync_copy(
            seq_ref.at[tc, pl.ds(qc * QUARTER, QUARTER)], idxs[r]
        )
        pltpu.sync_copy(pos_ref.at[tc], pbufs[r])
        for i in range(QUARTER // LANES):
            sl = pl.ds(i * LANES, LANES)
            v16 = idxs[r][sl]
            pidxs[r][sl] = lax.shift_left(
                lax.shift_right_logical(v16, CB_SH), CB_SH - 1
            ) | (v16 & (CB // 2 - 1))
        for h in range(2):
            pltpu.async_copy(
                table_ref.at[pidxs[r].at[pl.ds(h * 128, 128)]],
                gbufs[r].at[pl.ds(h * 128, 128)],
                gsems[r],
            )

    def wait_gathers(r):
        for h in range(2):
            pltpu.make_async_copy(
                table_ref.at[pidxs[r].at[pl.ds(h * 128, 128)]],
                gbufs[r].at[pl.ds(h * 128, 128)],
                gsems[r],
            ).wait()

    def scatter_copy(k, r):
        c = c_base + k
        tc = c >> 2
        qc = c & 3
        return pltpu.make_async_copy(
            tbufs[r],
            out_ref.at[tc, :, pl.ds(qc * QUARTER, QUARTER)],
            ssems[r],
        )

    colv = [(iota + k) & 15 for k in range(LANES)]

    def compute(r):
        gbuf = gbufs[r]
        tbuf = tbufs[r]
        pbuf = pbufs[r]

        @pl.loop(0, QUARTER // LANES)
        def _bc_loop(bc):
            sl = pl.ds(bc * LANES, LANES)
            v16 = idxs[r][sl]
            rowv = iota + bc * LANES
            hvec = lax.shift_left(lax.shift_right_logical(v16, CB_SH - 1) & 1, 6)
            for fb in range(D_MODEL // LANES):
                base = fb * LANES
                for kh in range(0, LANES, 8):
                    ks = range(kh, kh + 8)
                    featvs = [colv[k] + base for k in ks]
                    gvals = [
                        plsc.load_gather(gbuf, [rowv, fv + hvec])
                        for fv in featvs
                    ]
                    pvals = [
                        plsc.load_gather(
                            pbuf, [lax.shift_left(fv, 4) + iota]
                        )
                        for fv in featvs
                    ]
                    for fv, gv, pv in zip(featvs, gvals, pvals):
                        plsc.store_scatter(tbuf, [fv, rowv], gv + pv)

    stage(0, 0)
    stage(1, 1)
    wait_gathers(0)
    compute(0)
    scatter_copy(0, 0).start()

    stage(2, 0)
    wait_gathers(1)
    compute(1)
    scatter_copy(1, 1).start()

    @pl.loop(0, 11)
    def _main(p):
        for rr in range(2):
            k = 2 + 2 * p + rr
            q = rr
            stage(k + 1, 1 - q)
            wait_gathers(q)
            scatter_copy(k - 2, q).wait()
            compute(q)
            scatter_copy(k, q).start()

    wait_gathers(0)
    scatter_copy(22, 0).wait()
    compute(0)
    scatter_copy(24, 0).start()
    scatter_copy(23, 1).wait()
    scatter_copy(24, 0).wait()


def kernel(sequences, embedding_table):
    batch, seq_len = sequences.shape
    vocab, d_model = embedding_table.shape
    assert (batch, seq_len, d_model) == (BATCH, SEQ_LEN, D_MODEL)

    tt = jnp.transpose(embedding_table)
    table_p = _pack_table(tt, vocab)

    seq_t = jnp.transpose(sequences).astype(jnp.int32)
    pos = _positional_encoding(seq_len, d_model)
    pos_b = jnp.broadcast_to(
        pos[:, :, None], (seq_len, d_model, LANES)
    ).reshape(seq_len, d_model * LANES)

    mesh = plsc.VectorSubcoreMesh(core_axis_name="c", subcore_axis_name="s")
    out_p = pl.kernel(
        _gather_body,
        out_type=jax.ShapeDtypeStruct((seq_len, d_model, batch), jnp.float32),
        mesh=mesh,
        compiler_params=pltpu.CompilerParams(needs_layout_passes=False),
        scratch_types=[
            pltpu.VMEM((QUARTER,), jnp.int32),
            pltpu.VMEM((QUARTER,), jnp.int32),
            pltpu.VMEM((QUARTER,), jnp.int32),
            pltpu.VMEM((QUARTER,), jnp.int32),
            pltpu.VMEM((QUARTER, 128), jnp.float32),
            pltpu.VMEM((QUARTER, 128), jnp.float32),
            pltpu.VMEM((D_MODEL, QUARTER), jnp.float32),
            pltpu.VMEM((D_MODEL, QUARTER), jnp.float32),
            pltpu.VMEM((D_MODEL * LANES,), jnp.float32),
            pltpu.VMEM((D_MODEL * LANES,), jnp.float32),
        ]
        + [pltpu.SemaphoreType.DMA for _ in range(4)],
    )(seq_t, table_p, pos_b)
    return jnp.transpose(out_p, (2, 0, 1))


# speedup vs baseline: 2.0711x; 1.0417x over previous
import functools

import jax
import jax.numpy as jnp
from jax import lax
from jax.experimental import pallas as pl
from jax.experimental.pallas import tpu as pltpu
from jax.experimental.pallas import tpu_sc as plsc

D_MODEL = 64
LANES = 16
NUM_WORKERS = 32
BATCH = 1024
SEQ_LEN = 200
QUARTER = BATCH // 4
CB = 32768
CB_SH = 15


def _positional_encoding(max_len, d_model):
    depth = d_model // 2
    positions = jnp.arange(max_len, dtype=jnp.float32)[:, None]
    depths = jnp.arange(depth, dtype=jnp.float32)[None, :] / depth
    angle_rates = 1.0 / (10000.0 ** depths)
    angle_rads = positions * angle_rates
    return jnp.concatenate(
        [jnp.sin(angle_rads), jnp.cos(angle_rads)], axis=-1
    ).astype(jnp.float32)


def _tr_body(tt_ref, out_ref):
    x = tt_ref[...]
    scale = jnp.float32(8.0)
    y0 = jnp.transpose(x[:, : CB // 2]) * scale
    y1 = jnp.transpose(x[:, CB // 2 :]) * scale
    out_ref[...] = jnp.concatenate([y0, y1], axis=1)


def _pack_table(tt, vocab):
    grid = (vocab + CB - 1) // CB
    return pl.pallas_call(
        _tr_body,
        grid=(grid,),
        in_specs=[pl.BlockSpec((D_MODEL, CB), lambda i: (0, i))],
        out_specs=pl.BlockSpec((CB // 2, 128), lambda i: (i, 0)),
        out_shape=jax.ShapeDtypeStruct((grid * (CB // 2), 128), jnp.float32),
    )(tt)


def _gather_body(
    seq_ref, table_ref, pos_ref, out_ref,
    idx0, idx1, pidx0, pidx1, g0, g1, t0, t1, p0, p1,
    gs0, gs1, ss0, ss1,
):
    idxs = (idx0, idx1)
    pidxs = (pidx0, pidx1)
    gbufs = (g0, g1)
    tbufs = (t0, t1)
    pbufs = (p0, p1)
    gsems = (gs0, gs1)
    ssems = (ss0, ss1)

    nc = 2
    wid = lax.axis_index("s") * nc + lax.axis_index("c")
    chunks_per_w = (SEQ_LEN * 4) // NUM_WORKERS
    c_base = wid * chunks_per_w
    iota = lax.iota(jnp.int32, LANES)

    def stage(k, r):
        c = c_base + k
        tc = c >> 2
        qc = c & 3
        pltpu.sync_copy(
            seq_ref.at[tc, pl.ds(qc * QUARTER, QUARTER)], idxs[r]
        )
        pltpu.sync_copy(pos_ref.at[tc], pbufs[r])
        for i in range(QUARTER // LANES):
            sl = pl.ds(i * LANES, LANES)
            v16 = idxs[r][sl]
            pidxs[r][sl] = lax.shift_left(
                lax.shift_right_logical(v16, CB_SH), CB_SH - 1
            ) | (v16 & (CB // 2 - 1))
        for h in range(2):
            pltpu.async_copy(
                table_ref.at[pidxs[r].at[pl.ds(h * 128, 128)]],
                gbufs[r].at[pl.ds(h * 128, 128)],
                gsems[r],
            )

    def wait_gathers(r):
        for h in range(2):
            pltpu.make_async_copy(
                table_ref.at[pidxs[r].at[pl.ds(h * 128, 128)]],
                gbufs[r].at[pl.ds(h * 128, 128)],
                gsems[r],
            ).wait()

    def scatter_copy(k, r):
        c = c_base + k
        tc = c >> 2
        qc = c & 3
        return pltpu.make_async_copy(
            tbufs[r],
            out_ref.at[tc, :, pl.ds(qc * QUARTER, QUARTER)],
            ssems[r],
        )

    colv = [(iota + k) & 15 for k in range(LANES)]

    def compute(r):
        gbuf = gbufs[r]
        tbuf = tbufs[r]
        pbuf = pbufs[r]

        @pl.loop(0, QUARTER // LANES)
        def _bc_loop(bc):
            sl = pl.ds(bc * LANES, LANES)
            v16 = idxs[r][sl]
            rowv = iota + bc * LANES
            hvec = lax.shift_left(lax.shift_right_logical(v16, CB_SH - 1) & 1, 6)
            for fb in range(D_MODEL // LANES):
                base = fb * LANES
                for kh in range(0, LANES, 8):
                    ks = range(kh, kh + 8)
                    featvs = [colv[k] + base for k in ks]
                    gvals = [
                        plsc.load_gather(gbuf, [rowv, fv + hvec])
                        for fv in featvs
                    ]
                    pvals = [
                        plsc.load_gather(
                            pbuf, [lax.shift_left(fv, 4) + iota]
                        )
                        for fv in featvs
                    ]
                    for fv, gv, pv in zip(featvs, gvals, pvals):
                        plsc.store_scatter(tbuf, [fv, rowv], gv + pv)

    stage(0, 0)
    stage(1, 1)
    wait_gathers(0)
    compute(0)
    scatter_copy(0, 0).start()

    stage(2, 0)
    wait_gathers(1)
    compute(1)
    scatter_copy(1, 1).start()

    @pl.loop(0, 11)
    def _main(p):
        for rr in range(2):
            k = 2 + 2 * p + rr
            q = rr
            stage(k + 1, 1 - q)
            wait_gathers(q)
            scatter_copy(k - 2, q).wait()
            compute(q)
            scatter_copy(k, q).start()

    wait_gathers(0)
    scatter_copy(22, 0).wait()
    compute(0)
    scatter_copy(24, 0).start()
    scatter_copy(23, 1).wait()
    scatter_copy(24, 0).wait()


def kernel(sequences, embedding_table):
    batch, seq_len = sequences.shape
    vocab, d_model = embedding_table.shape
    assert (batch, seq_len, d_model) == (BATCH, SEQ_LEN, D_MODEL)

    tt = jnp.transpose(embedding_table)
    table_p = _pack_table(tt, vocab)

    seq_t = jnp.transpose(sequences).astype(jnp.int32)
    pos = _positional_encoding(seq_len, d_model)
    pos_b = jnp.broadcast_to(
        pos[:, :, None], (seq_len, d_model, LANES)
    ).reshape(seq_len, d_model * LANES)

    mesh = plsc.VectorSubcoreMesh(core_axis_name="c", subcore_axis_name="s")
    out_p = pl.kernel(
        _gather_body,
        out_type=jax.ShapeDtypeStruct((seq_len, d_model, batch), jnp.float32),
        mesh=mesh,
        compiler_params=pltpu.CompilerParams(needs_layout_passes=False),
        scratch_types=[
            pltpu.VMEM((QUARTER,), jnp.int32),
            pltpu.VMEM((QUARTER,), jnp.int32),
            pltpu.VMEM((QUARTER,), jnp.int32),
            pltpu.VMEM((QUARTER,), jnp.int32),
            pltpu.VMEM((QUARTER, 128), jnp.float32),
            pltpu.VMEM((QUARTER, 128), jnp.float32),
            pltpu.VMEM((D_MODEL, QUARTER), jnp.float32),
            pltpu.VMEM((D_MODEL, QUARTER), jnp.float32),
            pltpu.VMEM((D_MODEL * LANES,), jnp.float32),
            pltpu.VMEM((D_MODEL * LANES,), jnp.float32),
        ]
        + [pltpu.SemaphoreType.DMA for _ in range(4)],
    )(seq_t, table_p, pos_b)
    return jnp.transpose(out_p, (2, 0, 1))


# speedup vs baseline: 2.2343x; 1.0788x over previous
import functools

import jax
import jax.numpy as jnp
from jax import lax
from jax.experimental import pallas as pl
from jax.experimental.pallas import tpu as pltpu
from jax.experimental.pallas import tpu_sc as plsc

D_MODEL = 64
LANES = 16
NUM_WORKERS = 32
BATCH = 1024
SEQ_LEN = 200
QUARTER = BATCH // 4
CB = 32768
CB_SH = 15


def _positional_encoding(max_len, d_model):
    depth = d_model // 2
    positions = jnp.arange(max_len, dtype=jnp.float32)[:, None]
    depths = jnp.arange(depth, dtype=jnp.float32)[None, :] / depth
    angle_rates = 1.0 / (10000.0 ** depths)
    angle_rads = positions * angle_rates
    return jnp.concatenate(
        [jnp.sin(angle_rads), jnp.cos(angle_rads)], axis=-1
    ).astype(jnp.float32)


def _tr_body(tt_ref, out_ref):
    x = tt_ref[...]
    scale = jnp.float32(8.0)
    y0 = jnp.transpose(x[:, : CB // 2]) * scale
    y1 = jnp.transpose(x[:, CB // 2 :]) * scale
    out_ref[...] = jnp.concatenate([y0, y1], axis=1)


def _pack_table(tt, vocab):
    grid = (vocab + CB - 1) // CB
    return pl.pallas_call(
        _tr_body,
        grid=(grid,),
        in_specs=[pl.BlockSpec((D_MODEL, CB), lambda i: (0, i))],
        out_specs=pl.BlockSpec((CB // 2, 128), lambda i: (i, 0)),
        out_shape=jax.ShapeDtypeStruct((grid * (CB // 2), 128), jnp.float32),
    )(tt)


def _gather_body(
    seq_ref, table_ref, pos_ref, out_ref,
    idx_all, pidx_all, pos_all, g0, g1, t0b, t1b,
    gs0, gs1, ss0, ss1,
):
    gbufs = (g0, g1)
    tbufs = (t0b, t1b)
    gsems = (gs0, gs1)
    ssems = (ss0, ss1)

    nc = 2
    wid = lax.axis_index("s") * nc + lax.axis_index("c")
    chunks_per_w = (SEQ_LEN * 4) // NUM_WORKERS
    c_base = wid * chunks_per_w
    t0 = lax.shift_right_logical(c_base, 2)
    ta = pl.multiple_of(t0 & ~7, 8)
    iota = lax.iota(jnp.int32, LANES)

    pltpu.sync_copy(seq_ref.at[pl.ds(c_base * QUARTER, chunks_per_w * QUARTER)], idx_all)
    pltpu.sync_copy(pos_ref.at[pl.ds(ta, 16)], pos_all)

    @pl.loop(0, (chunks_per_w * QUARTER) // LANES, unroll=4)
    def _pidx_loop(i):
        sl = pl.ds(i * LANES, LANES)
        v16 = idx_all[sl]
        pidx_all[sl] = lax.shift_left(
            lax.shift_right_logical(v16, CB_SH), CB_SH - 1
        ) | (v16 & (CB // 2 - 1))

    def gather_copies(k, r):
        return [
            pltpu.make_async_copy(
                table_ref.at[pidx_all.at[pl.ds(k * QUARTER + h * 128, 128)]],
                gbufs[r].at[pl.ds(h * 128, 128)],
                gsems[r],
            )
            for h in range(2)
        ]

    def scatter_copy(k, r):
        c = c_base + k
        tc = lax.shift_right_logical(c, 2)
        qc = c & 3
        return pltpu.make_async_copy(
            tbufs[r],
            out_ref.at[tc, :, pl.ds(qc * QUARTER, QUARTER)],
            ssems[r],
        )

    colv = [(iota + k) & 15 for k in range(LANES)]

    def compute(k, r):
        gbuf = gbufs[r]
        tbuf = tbufs[r]
        c = c_base + k
        tloc = lax.shift_right_logical(c, 2) - ta
        trow = lax.broadcast(tloc, (LANES,))

        @pl.loop(0, QUARTER // LANES)
        def _bc_loop(bc):
            sl = pl.ds(k * QUARTER + bc * LANES, LANES)
            v16 = idx_all[sl]
            rowv = iota + bc * LANES
            hvec = lax.shift_left(lax.shift_right_logical(v16, CB_SH - 1) & 1, 6)
            for fb in range(D_MODEL // LANES):
                base = fb * LANES
                for kh in range(0, LANES, 8):
                    ks = range(kh, kh + 8)
                    featvs = [colv[q] + base for q in ks]
                    gvals = [
                        plsc.load_gather(gbuf, [rowv, fv + hvec])
                        for fv in featvs
                    ]
                    pvals = [
                        plsc.load_gather(pos_all, [trow, fv])
                        for fv in featvs
                    ]
                    for fv, gv, pv in zip(featvs, gvals, pvals):
                        plsc.store_scatter(tbuf, [fv, rowv], gv + pv)

    for cp in gather_copies(0, 0):
        cp.start()
    for cp in gather_copies(1, 1):
        cp.start()

    def step(k, r, fire_next, wait_scat):
        for cp in gather_copies(k, r):
            cp.wait()
        if wait_scat:
            scatter_copy(k - 2, r).wait()
        compute(k, r)
        if fire_next:
            for cp in gather_copies(k + 2, r):
                cp.start()
        scatter_copy(k, r).start()

    step(0, 0, True, False)
    step(1, 1, True, False)

    @pl.loop(0, 10)
    def _main(p):
        for rr in range(2):
            k = 2 + 2 * p + rr
            step(k, rr, True, True)

    step(22, 0, True, True)
    step(23, 1, False, True)
    step(24, 0, False, True)
    scatter_copy(23, 1).wait()
    scatter_copy(24, 0).wait()


def kernel(sequences, embedding_table):
    batch, seq_len = sequences.shape
    vocab, d_model = embedding_table.shape
    assert (batch, seq_len, d_model) == (BATCH, SEQ_LEN, D_MODEL)

    tt = jnp.transpose(embedding_table)
    table_p = _pack_table(tt, vocab)

    seq_t = jnp.transpose(sequences).astype(jnp.int32)
    seq_flat = seq_t.reshape(-1)
    pos = _positional_encoding(seq_len, d_model)
    pos_b = jnp.pad(pos, ((0, 8), (0, d_model)))

    mesh = plsc.VectorSubcoreMesh(core_axis_name="c", subcore_axis_name="s")
    out_p = pl.kernel(
        _gather_body,
        out_type=jax.ShapeDtypeStruct((seq_len, d_model, batch), jnp.float32),
        mesh=mesh,
        compiler_params=pltpu.CompilerParams(needs_layout_passes=False),
        scratch_types=[
            pltpu.VMEM((25 * QUARTER,), jnp.int32),
            pltpu.VMEM((25 * QUARTER,), jnp.int32),
            pltpu.VMEM((16, 128), jnp.float32),
            pltpu.VMEM((QUARTER, 128), jnp.float32),
            pltpu.VMEM((QUARTER, 128), jnp.float32),
            pltpu.VMEM((D_MODEL, QUARTER), jnp.float32),
            pltpu.VMEM((D_MODEL, QUARTER), jnp.float32),
        ]
        + [pltpu.SemaphoreType.DMA for _ in range(4)],
    )(seq_flat, table_p, pos_b)
    return jnp.transpose(out_p, (2, 0, 1))
